# Initial kernel scaffold; baseline (speedup 1.0000x reference)
import jax
import jax.numpy as jnp
from jax.experimental import pallas as pl


def kernel(token_ids, padding_mask, embeds):
    raise NotImplementedError("write your pallas kernel here")


# speedup vs baseline: 1.7160x; 1.7160x over previous
import functools

import jax
import jax.numpy as jnp
from jax import lax
from jax.experimental import pallas as pl
from jax.experimental.pallas import tpu as pltpu
from jax.experimental.pallas import tpu_sc as plsc

RATIO = 16
LANES = 16
NUM_WORKERS = 32


def _build_sc_kernel(batch, seq_len, hidden):
    num_global = seq_len // RATIO
    rows = batch * num_global
    rpw = rows // NUM_WORKERS
    wpw = rpw
    mesh = plsc.VectorSubcoreMesh(core_axis_name="c", subcore_axis_name="s")

    @functools.partial(
        pl.kernel,
        mesh=mesh,
        out_type=[
            jax.ShapeDtypeStruct((rows, hidden), jnp.float32),
            jax.ShapeDtypeStruct((rows,), jnp.int32),
        ],
        scratch_types=[
            pltpu.VMEM((hidden,), jnp.float32),
            pltpu.VMEM((hidden,), jnp.float32),
            pltpu.VMEM((RATIO * wpw,), jnp.int32),
            pltpu.VMEM((wpw,), jnp.int32),
            pltpu.SemaphoreType.DMA,
        ],
    )
    def sc_kernel(mask_hbm, embeds_hbm, out_hbm, gmask_hbm,
                  row0_v, row1_v, mask_v, gout_v, sem):
        wid = lax.axis_index("s") * 2 + lax.axis_index("c")
        base = wid * rpw

        pltpu.sync_copy(embeds_hbm.at[0], row0_v)
        pltpu.sync_copy(embeds_hbm.at[1], row1_v)

        for k in range(RATIO):
            pltpu.sync_copy(
                mask_hbm.at[pl.ds(k * rows + base, wpw)],
                mask_v.at[pl.ds(k * wpw, wpw)],
            )
        for g in range(wpw // LANES):
            acc = mask_v[pl.ds(g * LANES, LANES)]
            for k in range(1, RATIO):
                acc = jnp.minimum(acc, mask_v[pl.ds(k * wpw + g * LANES, LANES)])
            gout_v[pl.ds(g * LANES, LANES)] = acc
        pltpu.sync_copy(gout_v, gmask_hbm.at[pl.ds(base, wpw)])

        copies = [
            pltpu.async_copy(row0_v, out_hbm.at[base + i], sem)
            for i in range(rpw)
        ]
        for cp in copies:
            cp.wait()

        @pl.when(base % num_global == 0)
        def _():
            pltpu.sync_copy(row1_v, out_hbm.at[base])

    return sc_kernel


def kernel(token_ids, padding_mask, embeds):
    batch, seq_len = token_ids.shape
    hidden = embeds.shape[1]
    num_global = seq_len // RATIO
    mask_flat = (
        padding_mask.astype(jnp.int32)
        .reshape(batch * seq_len // RATIO, RATIO)
        .T.reshape(batch * seq_len)
    )
    out_flat, gmask = _build_sc_kernel(batch, seq_len, hidden)(mask_flat, embeds)
    out = out_flat.reshape(batch, num_global, hidden)
    gmask = gmask.reshape(batch, num_global).astype(jnp.bool_)
    return out, gmask


# speedup vs baseline: 2.1176x; 1.2340x over previous
import functools

import jax
import jax.numpy as jnp
from jax import lax
from jax.experimental import pallas as pl
from jax.experimental.pallas import tpu as pltpu
from jax.experimental.pallas import tpu_sc as plsc

RATIO = 16
LANES = 16
NUM_WORKERS = 32


def _build_sc_kernel(batch, seq_len, hidden):
    num_global = seq_len // RATIO
    rows = batch * num_global
    rpw = rows // NUM_WORKERS
    wpw = rpw
    mesh = plsc.VectorSubcoreMesh(core_axis_name="c", subcore_axis_name="s")

    @functools.partial(
        pl.kernel,
        mesh=mesh,
        out_type=[
            jax.ShapeDtypeStruct((rows, hidden), jnp.float32),
            jax.ShapeDtypeStruct((rows,), jnp.int32),
        ],
        scratch_types=[
            pltpu.VMEM((hidden,), jnp.float32),
            pltpu.VMEM((hidden,), jnp.float32),
            pltpu.VMEM((RATIO * wpw,), jnp.int32),
            pltpu.VMEM((wpw,), jnp.int32),
            pltpu.SemaphoreType.DMA,
            pltpu.SemaphoreType.DMA,
            pltpu.SemaphoreType.DMA,
            pltpu.SemaphoreType.DMA,
        ],
    )
    def sc_kernel(mask_hbm, embeds_hbm, out_hbm, gmask_hbm,
                  row0_v, row1_v, mask_v, gout_v,
                  sem_r0, sem_r1, sem_mask, sem_out):
        wid = lax.axis_index("s") * 2 + lax.axis_index("c")
        base = wid * rpw

        cp_r0 = pltpu.async_copy(embeds_hbm.at[0], row0_v, sem_r0)
        cp_r1 = pltpu.async_copy(embeds_hbm.at[1], row1_v, sem_r1)
        cp_mask = pltpu.async_copy(
            mask_hbm.at[pl.ds(wid * RATIO * wpw, RATIO * wpw)], mask_v,
            sem_mask)

        cp_r0.wait()
        out_copies = [
            pltpu.async_copy(row0_v, out_hbm.at[base + i], sem_out)
            for i in range(rpw)
        ]

        cp_mask.wait()
        for g in range(wpw // LANES):
            acc = mask_v[pl.ds(g * LANES, LANES)]
            for k in range(1, RATIO):
                acc = jnp.minimum(
                    acc, mask_v[pl.ds(k * wpw + g * LANES, LANES)])
            gout_v[pl.ds(g * LANES, LANES)] = acc
        pltpu.sync_copy(gout_v, gmask_hbm.at[pl.ds(base, wpw)])

        for cp in out_copies:
            cp.wait()

        cp_r1.wait()

        @pl.when(base % num_global == 0)
        def _():
            pltpu.sync_copy(row1_v, out_hbm.at[base])

    return sc_kernel


def kernel(token_ids, padding_mask, embeds):
    batch, seq_len = token_ids.shape
    hidden = embeds.shape[1]
    num_global = seq_len // RATIO
    rows = batch * seq_len // RATIO
    wpw = rows // NUM_WORKERS
    mask_t = (
        padding_mask.astype(jnp.int32)
        .reshape(NUM_WORKERS, wpw, RATIO)
        .transpose(0, 2, 1)
        .reshape(batch * seq_len)
    )
    out_flat, gmask = _build_sc_kernel(batch, seq_len, hidden)(mask_t, embeds)
    out = out_flat.reshape(batch, num_global, hidden)
    gmask = gmask.reshape(batch, num_global).astype(jnp.bool_)
    return out, gmask
